# baseline (speedup 1.0000x reference)
import functools

import jax
import jax.numpy as jnp
from jax import lax
from jax.experimental import pallas as pl
from jax.experimental.pallas import tpu as pltpu
from jax.experimental.pallas import tpu_sc as plsc

N = 10000
E = 320000
D = 128
CHP = 64
NPAD = 10240
NC = 2
NS = 16
NW = NC * NS
CHUNK = 128
K = 80
EW = K * CHUNK
EPAD = NW * EW
ROWS_PER_TILE = NPAD // NS


def _h_body(x_ref, wg_ref, bg_ref, h_ref):
    h_ref[...] = (
        jnp.dot(x_ref[...], wg_ref[...], preferred_element_type=jnp.float32)
        + bg_ref[...]
    )


def _compute_h(x_pad, Wg_pad, bg_pad):
    return pl.pallas_call(
        _h_body,
        grid=(NPAD // 1024,),
        in_specs=[
            pl.BlockSpec((1024, D), lambda i: (i, 0)),
            pl.BlockSpec((D, CHP), lambda i: (0, 0)),
            pl.BlockSpec((1, CHP), lambda i: (0, 0)),
        ],
        out_specs=pl.BlockSpec((1024, CHP), lambda i: (i, 0)),
        out_shape=jax.ShapeDtypeStruct((NPAD, CHP), jnp.float32),
    )(x_pad, Wg_pad, bg_pad)


def _agg_body(h_hbm, src_hbm, dst_hbm, zrow_hbm, out_hbm,
              src_v, dst_v, rows_a, rows_b, agg_sh, sem_a, sem_b):
    c = lax.axis_index("c")
    s = lax.axis_index("s")
    wid = s * NC + c
    row0 = s * ROWS_PER_TILE

    pltpu.sync_copy(zrow_hbm, agg_sh.at[pl.ds(row0, ROWS_PER_TILE)])
    pltpu.sync_copy(src_hbm.at[wid], src_v)
    pltpu.sync_copy(dst_hbm.at[wid], dst_v)
    plsc.subcore_barrier()

    def step(jj, carry):
        j = jj * 2
        ca = pltpu.async_copy(h_hbm.at[src_v.at[j]], rows_a, sem_a)
        cb = pltpu.async_copy(h_hbm.at[src_v.at[j + 1]], rows_b, sem_b)
        ca.wait()
        pltpu.sync_copy(rows_a, agg_sh.at[dst_v.at[j]], add=True)
        cb.wait()
        pltpu.sync_copy(rows_b, agg_sh.at[dst_v.at[j + 1]], add=True)
        return carry

    lax.fori_loop(0, K // 2, step, 0, unroll=False)

    plsc.subcore_barrier()
    pltpu.sync_copy(
        agg_sh.at[pl.ds(row0, ROWS_PER_TILE)],
        out_hbm.at[c, pl.ds(row0, ROWS_PER_TILE)],
    )


@functools.partial(
    pl.kernel,
    out_type=jax.ShapeDtypeStruct((NC, NPAD, CHP), jnp.float32),
    mesh=plsc.VectorSubcoreMesh(
        core_axis_name="c", subcore_axis_name="s", num_cores=NC, num_subcores=NS
    ),
    scratch_types=[
        pltpu.VMEM((K, CHUNK), jnp.int32),
        pltpu.VMEM((K, CHUNK), jnp.int32),
        pltpu.VMEM((CHUNK, CHP), jnp.float32),
        pltpu.VMEM((CHUNK, CHP), jnp.float32),
        pltpu.MemorySpace.VMEM_SHARED((NPAD, CHP), jnp.float32),
        pltpu.SemaphoreType.DMA,
        pltpu.SemaphoreType.DMA,
    ],
    compiler_params=pltpu.CompilerParams(use_tc_tiling_on_sc=False),
)
def _aggregate(h_hbm, src_hbm, dst_hbm, zrow_hbm, out_hbm,
               src_v, dst_v, rows_a, rows_b, agg_sh, sem_a, sem_b):
    _agg_body(h_hbm, src_hbm, dst_hbm, zrow_hbm, out_hbm,
              src_v, dst_v, rows_a, rows_b, agg_sh, sem_a, sem_b)


def _swish(z):
    return z * jax.nn.sigmoid(z)


def _mlp_body(p_ref, gamma_ref, beta_ref, alpha_ref,
              w1, b1, w2, b2, w3, b3, w31, b31, w32, b32,
              wv, bv, w4, b4, w41, b41, w5, b5, out_ref):
    agg = p_ref[0] + p_ref[1]
    a = gamma_ref[...] * agg + beta_ref[...]
    h = jnp.where(a > 0, a, alpha_ref[...] * a)

    def dense(v, w, b):
        return jnp.dot(v, w[...], preferred_element_type=jnp.float32) + b[...]

    h = _swish(dense(h, w1, b1))
    h = dense(h, w2, b2)
    h = _swish(dense(h, w3, b3))
    h = dense(h, w31, b31)
    h = _swish(dense(h, w32, b32))
    h = dense(h, wv, bv)
    h = dense(h, w4, b4)
    h = dense(h, w41, b41)
    h = dense(h, w5, b5)
    out_ref[...] = jnp.round(h * 1000.0) / 1000.0


def _mlp(partials, gamma_pad, beta_pad, alpha_pad, weights):
    full = lambda shape: pl.BlockSpec(shape, lambda i: tuple(0 for _ in shape))
    w_specs = []
    for w in weights:
        w_specs.append(full(w.shape))
    return pl.pallas_call(
        _mlp_body,
        grid=(NPAD // 1024,),
        in_specs=[
            pl.BlockSpec((NC, 1024, CHP), lambda i: (0, i, 0)),
            full((1, CHP)), full((1, CHP)), full((1, CHP)),
            *w_specs,
        ],
        out_specs=pl.BlockSpec((1024, 1), lambda i: (i, 0)),
        out_shape=jax.ShapeDtypeStruct((NPAD, 1), jnp.float32),
    )(partials, gamma_pad, beta_pad, alpha_pad, *weights)


def kernel(x, edge_index, i, Wg, bg, gamma, beta, alpha,
           W1, b1, W2, b2, W3, b3, W31, b31, W32, b32,
           Wv, bv, W4, b4, W41, b41, W5, b5):
    del i

    x_pad = jnp.pad(x, ((0, NPAD - N), (0, 0)))
    Wg_pad = jnp.pad(Wg, ((0, 0), (0, CHP - Wg.shape[1])))
    bg_pad = jnp.pad(bg, (0, CHP - bg.shape[0])).reshape(1, CHP)
    h_pad = _compute_h(x_pad, Wg_pad, bg_pad)

    src = edge_index[0].astype(jnp.int32)
    dst = edge_index[1].astype(jnp.int32)
    src3 = jnp.pad(src, (0, EPAD - E)).reshape(NW, K, CHUNK)
    dst3 = jnp.pad(dst, (0, EPAD - E),
                   constant_values=NPAD - 1).reshape(NW, K, CHUNK)
    zrow = jnp.zeros((ROWS_PER_TILE, CHP), jnp.float32)
    partials = _aggregate(h_pad, src3, dst3, zrow)

    pad_vec = lambda v, cv=0.0: jnp.pad(
        v, (0, CHP - v.shape[0]), constant_values=cv).reshape(1, CHP)
    gamma_pad = pad_vec(gamma)
    beta_pad = pad_vec(beta)
    alpha_pad = pad_vec(alpha)
    W1_pad = jnp.pad(W1, ((0, CHP - W1.shape[0]), (0, 0)))
    weights = [W1_pad, b1.reshape(1, -1), W2, b2.reshape(1, -1),
               W3, b3.reshape(1, -1), W31, b31.reshape(1, -1),
               W32, b32.reshape(1, -1), Wv, bv.reshape(1, -1),
               W4, b4.reshape(1, -1), W41, b41.reshape(1, -1),
               W5, b5.reshape(1, -1)]
    out = _mlp(partials, gamma_pad, beta_pad, alpha_pad, weights)
    return out[:N]


# speedup vs baseline: 1.9219x; 1.9219x over previous
import functools

import jax
import jax.numpy as jnp
from jax import lax
from jax.experimental import pallas as pl
from jax.experimental.pallas import tpu as pltpu
from jax.experimental.pallas import tpu_sc as plsc

N = 10000
E = 320000
D = 128
CHP = 64
NPAD = 10240
NC = 2
NS = 16
NW = NC * NS
CHUNK = 125
K = 80
EW = K * CHUNK
ROWS_PER_TILE = NPAD // NS


def _h_body(x_ref, wg_ref, bg_ref, h_ref):
    h_ref[...] = (
        jnp.dot(x_ref[...], wg_ref[...], preferred_element_type=jnp.float32)
        + bg_ref[...]
    )


def _compute_h(x, Wg_pad, bg_pad):
    return pl.pallas_call(
        _h_body,
        grid=(NPAD // 1024,),
        in_specs=[
            pl.BlockSpec((1024, D), lambda i: (i, 0)),
            pl.BlockSpec((D, CHP), lambda i: (0, 0)),
            pl.BlockSpec((1, CHP), lambda i: (0, 0)),
        ],
        out_specs=pl.BlockSpec((1024, CHP), lambda i: (i, 0)),
        out_shape=jax.ShapeDtypeStruct((NPAD, CHP), jnp.float32),
    )(x, Wg_pad, bg_pad)


def _agg_body(h_hbm, src_hbm, dst_hbm, zrow_hbm, out_hbm,
              src_v, dst_v, rows_a, rows_b, h_sh, agg_sh, sem_a, sem_b):
    c = lax.axis_index("c")
    s = lax.axis_index("s")
    wid = s * NC + c
    row0 = s * ROWS_PER_TILE

    pltpu.sync_copy(zrow_hbm, agg_sh.at[pl.ds(row0, ROWS_PER_TILE)])
    pltpu.sync_copy(h_hbm.at[pl.ds(row0, ROWS_PER_TILE)],
                    h_sh.at[pl.ds(row0, ROWS_PER_TILE)])
    pltpu.sync_copy(src_hbm.at[wid], src_v)
    pltpu.sync_copy(dst_hbm.at[wid], dst_v)
    plsc.subcore_barrier()

    def step(jj, carry):
        j = jj * 2
        ca = pltpu.async_copy(h_sh.at[src_v.at[j]], rows_a, sem_a)
        cb = pltpu.async_copy(h_sh.at[src_v.at[j + 1]], rows_b, sem_b)
        ca.wait()
        pltpu.sync_copy(rows_a, agg_sh.at[dst_v.at[j]], add=True)
        cb.wait()
        pltpu.sync_copy(rows_b, agg_sh.at[dst_v.at[j + 1]], add=True)
        return carry

    lax.fori_loop(0, K // 2, step, 0, unroll=False)

    plsc.subcore_barrier()
    pltpu.sync_copy(
        agg_sh.at[pl.ds(row0, ROWS_PER_TILE)],
        out_hbm.at[c, pl.ds(row0, ROWS_PER_TILE)],
    )


@functools.partial(
    pl.kernel,
    out_type=jax.ShapeDtypeStruct((NC, NPAD, CHP), jnp.float32),
    mesh=plsc.VectorSubcoreMesh(
        core_axis_name="c", subcore_axis_name="s", num_cores=NC, num_subcores=NS
    ),
    scratch_types=[
        pltpu.VMEM((K, CHUNK), jnp.int32),
        pltpu.VMEM((K, CHUNK), jnp.int32),
        pltpu.VMEM((CHUNK, CHP), jnp.float32),
        pltpu.VMEM((CHUNK, CHP), jnp.float32),
        pltpu.MemorySpace.VMEM_SHARED((NPAD, CHP), jnp.float32),
        pltpu.MemorySpace.VMEM_SHARED((NPAD, CHP), jnp.float32),
        pltpu.SemaphoreType.DMA,
        pltpu.SemaphoreType.DMA,
    ],
    compiler_params=pltpu.CompilerParams(use_tc_tiling_on_sc=False),
)
def _aggregate(h_hbm, src_hbm, dst_hbm, zrow_hbm, out_hbm,
               src_v, dst_v, rows_a, rows_b, h_sh, agg_sh, sem_a, sem_b):
    _agg_body(h_hbm, src_hbm, dst_hbm, zrow_hbm, out_hbm,
              src_v, dst_v, rows_a, rows_b, h_sh, agg_sh, sem_a, sem_b)


def _swish(z):
    return z * jax.nn.sigmoid(z)


def _mlp_body(p_ref, gamma_ref, beta_ref, alpha_ref,
              w1, b1, w2, b2, w3, b3, w31, b31, w32, b32,
              wv, bv, w4, b4, w41, b41, w5, b5, out_ref):
    agg = p_ref[0] + p_ref[1]
    a = gamma_ref[...] * agg + beta_ref[...]
    h = jnp.where(a > 0, a, alpha_ref[...] * a)

    def dense(v, w, b):
        return jnp.dot(v, w[...], preferred_element_type=jnp.float32) + b[...]

    h = _swish(dense(h, w1, b1))
    h = dense(h, w2, b2)
    h = _swish(dense(h, w3, b3))
    h = dense(h, w31, b31)
    h = _swish(dense(h, w32, b32))
    h = dense(h, wv, bv)
    h = dense(h, w4, b4)
    h = dense(h, w41, b41)
    h = dense(h, w5, b5)
    out_ref[...] = jnp.round(h * 1000.0) / 1000.0


def _mlp(partials, gamma_pad, beta_pad, alpha_pad, weights):
    full = lambda shape: pl.BlockSpec(shape, lambda i: tuple(0 for _ in shape))
    w_specs = []
    for w in weights:
        w_specs.append(full(w.shape))
    return pl.pallas_call(
        _mlp_body,
        grid=(NPAD // 1024,),
        in_specs=[
            pl.BlockSpec((NC, 1024, CHP), lambda i: (0, i, 0)),
            full((1, CHP)), full((1, CHP)), full((1, CHP)),
            *w_specs,
        ],
        out_specs=pl.BlockSpec((1024, 1), lambda i: (i, 0)),
        out_shape=jax.ShapeDtypeStruct((NPAD, 1), jnp.float32),
    )(partials, gamma_pad, beta_pad, alpha_pad, *weights)


def kernel(x, edge_index, i, Wg, bg, gamma, beta, alpha,
           W1, b1, W2, b2, W3, b3, W31, b31, W32, b32,
           Wv, bv, W4, b4, W41, b41, W5, b5):
    del i

    Wg_pad = jnp.pad(Wg, ((0, 0), (0, CHP - Wg.shape[1])))
    bg_pad = jnp.pad(bg, (0, CHP - bg.shape[0])).reshape(1, CHP)
    h_pad = _compute_h(x, Wg_pad, bg_pad)

    src3 = edge_index[0].astype(jnp.int32).reshape(NW, K, CHUNK)
    dst3 = edge_index[1].astype(jnp.int32).reshape(NW, K, CHUNK)
    zrow = jnp.zeros((ROWS_PER_TILE, CHP), jnp.float32)
    partials = _aggregate(h_pad, src3, dst3, zrow)

    pad_vec = lambda v, cv=0.0: jnp.pad(
        v, (0, CHP - v.shape[0]), constant_values=cv).reshape(1, CHP)
    gamma_pad = pad_vec(gamma)
    beta_pad = pad_vec(beta)
    alpha_pad = pad_vec(alpha)
    W1_pad = jnp.pad(W1, ((0, CHP - W1.shape[0]), (0, 0)))
    weights = [W1_pad, b1.reshape(1, -1), W2, b2.reshape(1, -1),
               W3, b3.reshape(1, -1), W31, b31.reshape(1, -1),
               W32, b32.reshape(1, -1), Wv, bv.reshape(1, -1),
               W4, b4.reshape(1, -1), W41, b41.reshape(1, -1),
               W5, b5.reshape(1, -1)]
    out = _mlp(partials, gamma_pad, beta_pad, alpha_pad, weights)
    return out[:N]


# speedup vs baseline: 2.5731x; 1.3388x over previous
import functools

import jax
import jax.numpy as jnp
from jax import lax
from jax.experimental import pallas as pl
from jax.experimental.pallas import tpu as pltpu
from jax.experimental.pallas import tpu_sc as plsc

N = 10000
E = 320000
D = 128
CHP = 64
NPAD = 10240
NC = 2
NS = 16
NW = NC * NS
CHUNK = 125
K = 80
EW = K * CHUNK
ROWS_PER_TILE = NPAD // NS
NB = 8
LEAD = 4


def _h_body(x_ref, wg_ref, bg_ref, h_ref):
    h_ref[...] = (
        jnp.dot(x_ref[...], wg_ref[...], preferred_element_type=jnp.float32)
        + bg_ref[...]
    )


def _compute_h(x, Wg_pad, bg_pad):
    return pl.pallas_call(
        _h_body,
        grid=(NPAD // 1024,),
        in_specs=[
            pl.BlockSpec((1024, D), lambda i: (i, 0)),
            pl.BlockSpec((D, CHP), lambda i: (0, 0)),
            pl.BlockSpec((1, CHP), lambda i: (0, 0)),
        ],
        out_specs=pl.BlockSpec((1024, CHP), lambda i: (i, 0)),
        out_shape=jax.ShapeDtypeStruct((NPAD, CHP), jnp.float32),
    )(x, Wg_pad, bg_pad)


def _agg_body(h_hbm, src_hbm, dst_hbm, zrow_hbm, out_hbm,
              src_v, dst_v, rows, agg_sh, *sems):
    gsem = sems[:NB]
    asem = sems[NB:]
    c = lax.axis_index("c")
    s = lax.axis_index("s")
    wid = s * NC + c
    row0 = s * ROWS_PER_TILE

    pltpu.sync_copy(zrow_hbm, agg_sh.at[pl.ds(row0, ROWS_PER_TILE)])
    pltpu.sync_copy(src_hbm.at[wid], src_v)
    pltpu.sync_copy(dst_hbm.at[wid], dst_v)
    plsc.subcore_barrier()

    def start_gather(j, b):
        pltpu.async_copy(h_hbm.at[src_v.at[j]], rows.at[b], gsem[b])

    def wait_gather(j, b):
        pltpu.make_async_copy(h_hbm.at[src_v.at[j]], rows.at[b],
                              gsem[b]).wait()

    def start_add(j, b):
        pltpu.async_copy(rows.at[b], agg_sh.at[dst_v.at[j]], asem[b],
                         add=True)

    def wait_add(j, b):
        pltpu.make_async_copy(rows.at[b], agg_sh.at[dst_v.at[j]],
                              asem[b]).wait()

    for j in range(LEAD):
        start_gather(j, j % NB)
    for j in range(LEAD):
        wait_gather(j, j % NB)
        start_add(j, j % NB)
        start_gather(j + LEAD, (j + LEAD) % NB)

    def outer(g, carry):
        j0 = LEAD + g * NB
        for u in range(NB):
            j = j0 + u
            b = (LEAD + u) % NB
            b2 = u
            wait_gather(j, b)
            start_add(j, b)
            wait_add(j - LEAD, b2)
            start_gather(j + LEAD, b2)
        return carry

    lax.fori_loop(0, (K - 2 * LEAD) // NB, outer, 0, unroll=False)

    for j in range(K - LEAD, K):
        b = j % NB
        wait_gather(j, b)
        start_add(j, b)
    for b in range(NB):
        wait_add(K - NB + b, b)

    plsc.subcore_barrier()
    pltpu.sync_copy(
        agg_sh.at[pl.ds(row0, ROWS_PER_TILE)],
        out_hbm.at[c, pl.ds(row0, ROWS_PER_TILE)],
    )


@functools.partial(
    pl.kernel,
    out_type=jax.ShapeDtypeStruct((NC, NPAD, CHP), jnp.float32),
    mesh=plsc.VectorSubcoreMesh(
        core_axis_name="c", subcore_axis_name="s", num_cores=NC, num_subcores=NS
    ),
    scratch_types=[
        pltpu.VMEM((K, CHUNK), jnp.int32),
        pltpu.VMEM((K, CHUNK), jnp.int32),
        pltpu.VMEM((NB, CHUNK, CHP), jnp.float32),
        pltpu.MemorySpace.VMEM_SHARED((NPAD, CHP), jnp.float32),
    ] + [pltpu.SemaphoreType.DMA] * (2 * NB),
    compiler_params=pltpu.CompilerParams(use_tc_tiling_on_sc=False),
)
def _aggregate(*args):
    _agg_body(*args)


def _swish(z):
    return z * jax.nn.sigmoid(z)


def _mlp_body(p_ref, gamma_ref, beta_ref, alpha_ref,
              w1, b1, w2, b2, w3, b3, w31, b31, w32, b32,
              wv, bv, w4, b4, w41, b41, w5, b5, out_ref):
    agg = p_ref[0] + p_ref[1]
    a = gamma_ref[...] * agg + beta_ref[...]
    h = jnp.where(a > 0, a, alpha_ref[...] * a)

    def dense(v, w, b):
        return jnp.dot(v, w[...], preferred_element_type=jnp.float32) + b[...]

    h = _swish(dense(h, w1, b1))
    h = dense(h, w2, b2)
    h = _swish(dense(h, w3, b3))
    h = dense(h, w31, b31)
    h = _swish(dense(h, w32, b32))
    h = dense(h, wv, bv)
    h = dense(h, w4, b4)
    h = dense(h, w41, b41)
    h = dense(h, w5, b5)
    out_ref[...] = jnp.round(h * 1000.0) / 1000.0


def _mlp(partials, gamma_pad, beta_pad, alpha_pad, weights):
    full = lambda shape: pl.BlockSpec(shape, lambda i: tuple(0 for _ in shape))
    w_specs = []
    for w in weights:
        w_specs.append(full(w.shape))
    return pl.pallas_call(
        _mlp_body,
        grid=(NPAD // 1024,),
        in_specs=[
            pl.BlockSpec((NC, 1024, CHP), lambda i: (0, i, 0)),
            full((1, CHP)), full((1, CHP)), full((1, CHP)),
            *w_specs,
        ],
        out_specs=pl.BlockSpec((1024, 1), lambda i: (i, 0)),
        out_shape=jax.ShapeDtypeStruct((NPAD, 1), jnp.float32),
    )(partials, gamma_pad, beta_pad, alpha_pad, *weights)


def kernel(x, edge_index, i, Wg, bg, gamma, beta, alpha,
           W1, b1, W2, b2, W3, b3, W31, b31, W32, b32,
           Wv, bv, W4, b4, W41, b41, W5, b5):
    del i

    Wg_pad = jnp.pad(Wg, ((0, 0), (0, CHP - Wg.shape[1])))
    bg_pad = jnp.pad(bg, (0, CHP - bg.shape[0])).reshape(1, CHP)
    h_pad = _compute_h(x, Wg_pad, bg_pad)

    src3 = edge_index[0].astype(jnp.int32).reshape(NW, K, CHUNK)
    dst3 = edge_index[1].astype(jnp.int32).reshape(NW, K, CHUNK)
    zrow = jnp.zeros((ROWS_PER_TILE, CHP), jnp.float32)
    partials = _aggregate(h_pad, src3, dst3, zrow)

    pad_vec = lambda v, cv=0.0: jnp.pad(
        v, (0, CHP - v.shape[0]), constant_values=cv).reshape(1, CHP)
    gamma_pad = pad_vec(gamma)
    beta_pad = pad_vec(beta)
    alpha_pad = pad_vec(alpha)
    W1_pad = jnp.pad(W1, ((0, CHP - W1.shape[0]), (0, 0)))
    weights = [W1_pad, b1.reshape(1, -1), W2, b2.reshape(1, -1),
               W3, b3.reshape(1, -1), W31, b31.reshape(1, -1),
               W32, b32.reshape(1, -1), Wv, bv.reshape(1, -1),
               W4, b4.reshape(1, -1), W41, b41.reshape(1, -1),
               W5, b5.reshape(1, -1)]
    out = _mlp(partials, gamma_pad, beta_pad, alpha_pad, weights)
    return out[:N]
